# Initial kernel scaffold; baseline (speedup 1.0000x reference)
import jax
import jax.numpy as jnp
from jax.experimental import pallas as pl


def kernel(x, edge_index, Wl1, Wr1, b1, Wl2, Wr2, b2, Wl3, Wr3, b3):
    raise NotImplementedError("write your pallas kernel here")


# speedup vs baseline: 1.0371x; 1.0371x over previous
import functools

import jax
import jax.numpy as jnp
from jax.experimental import pallas as pl
from jax.experimental.pallas import tpu as pltpu

N_NODES = 10000
D_FEAT = 128


def _layer_body(act, agg_ref, deg_ref, h_ref, wl_ref, wr_ref, b_ref, o_ref):
    deg = jnp.maximum(deg_ref[...], 1.0)
    agg = agg_ref[...] / deg
    out = (jnp.dot(agg, wl_ref[...], preferred_element_type=jnp.float32)
           + jnp.dot(h_ref[...], wr_ref[...], preferred_element_type=jnp.float32)
           + b_ref[...][None, :])
    if act == "relu":
        out = jnp.maximum(out, 0.0)
    elif act == "sigmoid":
        out = jax.nn.sigmoid(out)
    o_ref[...] = out


def _tc_layer(agg_sum, deg, h, Wl, Wr, b, act):
    n, d_out = h.shape[0], Wl.shape[1]
    blk = 2000
    grid = (n // blk,)
    return pl.pallas_call(
        functools.partial(_layer_body, act),
        grid=grid,
        in_specs=[
            pl.BlockSpec((blk, agg_sum.shape[1]), lambda i: (i, 0)),
            pl.BlockSpec((blk, 1), lambda i: (i, 0)),
            pl.BlockSpec((blk, h.shape[1]), lambda i: (i, 0)),
            pl.BlockSpec(Wl.shape, lambda i: (0, 0)),
            pl.BlockSpec(Wr.shape, lambda i: (0, 0)),
            pl.BlockSpec(b.shape, lambda i: (0,)),
        ],
        out_specs=pl.BlockSpec((blk, d_out), lambda i: (i, 0)),
        out_shape=jax.ShapeDtypeStruct((n, d_out), jnp.float32),
    )(agg_sum, deg, h, Wl, Wr, b)


def kernel(x, edge_index, Wl1, Wr1, b1, Wl2, Wr2, b2, Wl3, Wr3, b3):
    src = edge_index[0].astype(jnp.int32)
    dst = edge_index[1].astype(jnp.int32)
    deg = jax.ops.segment_sum(jnp.ones((src.shape[0],), jnp.float32), dst,
                              num_segments=N_NODES)[:, None]

    def agg_of(h):
        msg = jnp.take(h, src, axis=0)
        return jax.ops.segment_sum(msg, dst, num_segments=N_NODES)

    h1 = _tc_layer(agg_of(x), deg, x, Wl1, Wr1, b1, "relu")
    h2 = _tc_layer(agg_of(h1), deg, h1, Wl2, Wr2, b2, "relu")
    return _tc_layer(agg_of(h2), deg, h2, Wl3, Wr3, b3, "sigmoid")


# speedup vs baseline: 6.9340x; 6.6857x over previous
import functools

import jax
import jax.numpy as jnp
from jax import lax
from jax.experimental import pallas as pl
from jax.experimental.pallas import tpu as pltpu
from jax.experimental.pallas import tpu_sc as plsc

N_NODES = 10000
N_PAD = 10240
N_EDGES = 320000
D_FEAT = 128
NC = 2
NS = 16
NW = NC * NS
EDGES_PER_TILE = N_EDGES // NW
CHUNK = 80
N_CHUNKS = EDGES_PER_TILE // CHUNK
ROWS_PER_TILE = N_PAD // NS


def _sc_aggregate(h, src3, dst3, zeros):
    mesh = plsc.VectorSubcoreMesh(core_axis_name="c", subcore_axis_name="s")

    @functools.partial(
        pl.kernel,
        mesh=mesh,
        out_type=jax.ShapeDtypeStruct((NC, N_PAD, D_FEAT), jnp.float32),
        scratch_types=[
            pltpu.VMEM((N_CHUNKS, CHUNK), jnp.int32),
            pltpu.VMEM((N_CHUNKS, CHUNK), jnp.int32),
            pltpu.VMEM((CHUNK, D_FEAT), jnp.float32),
            pltpu.VMEM_SHARED((N_PAD, D_FEAT), jnp.float32),
        ],
    )
    def k(h_hbm, src_hbm, dst_hbm, z_hbm, out_hbm, src_v, dst_v, rows_v, acc_sh):
        c = lax.axis_index("c")
        s = lax.axis_index("s")
        wid = c * NS + s
        row0 = s * ROWS_PER_TILE
        pltpu.sync_copy(z_hbm.at[pl.ds(row0, ROWS_PER_TILE)],
                        acc_sh.at[pl.ds(row0, ROWS_PER_TILE)])
        pltpu.sync_copy(src_hbm.at[wid], src_v)
        pltpu.sync_copy(dst_hbm.at[wid], dst_v)
        plsc.subcore_barrier()

        @pl.loop(0, N_CHUNKS)
        def _(j):
            pltpu.sync_copy(h_hbm.at[src_v.at[j]], rows_v)
            pltpu.sync_copy(rows_v, acc_sh.at[dst_v.at[j]], add=True)

        plsc.subcore_barrier()
        pltpu.sync_copy(acc_sh.at[pl.ds(row0, ROWS_PER_TILE)],
                        out_hbm.at[c, pl.ds(row0, ROWS_PER_TILE)])

    return k(h, src3, dst3, zeros)


def _sc_degree(ones, dst3, zeros):
    mesh = plsc.VectorSubcoreMesh(core_axis_name="c", subcore_axis_name="s")

    @functools.partial(
        pl.kernel,
        mesh=mesh,
        out_type=jax.ShapeDtypeStruct((NC, N_PAD, D_FEAT), jnp.float32),
        scratch_types=[
            pltpu.VMEM((N_CHUNKS, CHUNK), jnp.int32),
            pltpu.VMEM((CHUNK, D_FEAT), jnp.float32),
            pltpu.VMEM_SHARED((N_PAD, D_FEAT), jnp.float32),
        ],
    )
    def k(ones_hbm, dst_hbm, z_hbm, out_hbm, dst_v, ones_v, acc_sh):
        c = lax.axis_index("c")
        s = lax.axis_index("s")
        wid = c * NS + s
        row0 = s * ROWS_PER_TILE
        pltpu.sync_copy(z_hbm.at[pl.ds(row0, ROWS_PER_TILE)],
                        acc_sh.at[pl.ds(row0, ROWS_PER_TILE)])
        pltpu.sync_copy(ones_hbm, ones_v)
        pltpu.sync_copy(dst_hbm.at[wid], dst_v)
        plsc.subcore_barrier()

        @pl.loop(0, N_CHUNKS)
        def _(j):
            pltpu.sync_copy(ones_v, acc_sh.at[dst_v.at[j]], add=True)

        plsc.subcore_barrier()
        pltpu.sync_copy(acc_sh.at[pl.ds(row0, ROWS_PER_TILE)],
                        out_hbm.at[c, pl.ds(row0, ROWS_PER_TILE)])

    return k(ones, dst3, zeros)


def _layer1_body(p_ref, hist_ref, x_ref, wl_ref, wr_ref, b_ref, o_ref, deg_ref):
    deg = jnp.maximum(hist_ref[0, :, :1] + hist_ref[1, :, :1], 1.0)
    agg = (p_ref[0] + p_ref[1]) / deg
    out = (jnp.dot(agg, wl_ref[...], preferred_element_type=jnp.float32)
           + jnp.dot(x_ref[...], wr_ref[...], preferred_element_type=jnp.float32)
           + b_ref[...][None, :])
    o_ref[...] = jnp.maximum(out, 0.0)
    deg_ref[...] = deg


def _layerN_body(act, p_ref, deg_ref, h_ref, wl_ref, wr_ref, b_ref, o_ref):
    agg = (p_ref[0] + p_ref[1]) / deg_ref[...]
    out = (jnp.dot(agg, wl_ref[...], preferred_element_type=jnp.float32)
           + jnp.dot(h_ref[...], wr_ref[...], preferred_element_type=jnp.float32)
           + b_ref[...][None, :])
    if act == "relu":
        out = jnp.maximum(out, 0.0)
    else:
        out = jax.nn.sigmoid(out)
    o_ref[...] = out


_BLK = 2048


def _tc_layer1(p, hist, x, Wl, Wr, b):
    d_out = Wl.shape[1]
    return pl.pallas_call(
        _layer1_body,
        grid=(N_PAD // _BLK,),
        in_specs=[
            pl.BlockSpec((NC, _BLK, D_FEAT), lambda i: (0, i, 0)),
            pl.BlockSpec((NC, _BLK, D_FEAT), lambda i: (0, i, 0)),
            pl.BlockSpec((_BLK, D_FEAT), lambda i: (i, 0)),
            pl.BlockSpec(Wl.shape, lambda i: (0, 0)),
            pl.BlockSpec(Wr.shape, lambda i: (0, 0)),
            pl.BlockSpec(b.shape, lambda i: (0,)),
        ],
        out_specs=[
            pl.BlockSpec((_BLK, d_out), lambda i: (i, 0)),
            pl.BlockSpec((_BLK, 1), lambda i: (i, 0)),
        ],
        out_shape=[
            jax.ShapeDtypeStruct((N_PAD, d_out), jnp.float32),
            jax.ShapeDtypeStruct((N_PAD, 1), jnp.float32),
        ],
    )(p, hist, x, Wl, Wr, b)


def _tc_layerN(p, deg, h, Wl, Wr, b, act):
    d_out = Wl.shape[1]
    return pl.pallas_call(
        functools.partial(_layerN_body, act),
        grid=(N_PAD // _BLK,),
        in_specs=[
            pl.BlockSpec((NC, _BLK, D_FEAT), lambda i: (0, i, 0)),
            pl.BlockSpec((_BLK, 1), lambda i: (i, 0)),
            pl.BlockSpec((_BLK, D_FEAT), lambda i: (i, 0)),
            pl.BlockSpec(Wl.shape, lambda i: (0, 0)),
            pl.BlockSpec(Wr.shape, lambda i: (0, 0)),
            pl.BlockSpec(b.shape, lambda i: (0,)),
        ],
        out_specs=pl.BlockSpec((_BLK, d_out), lambda i: (i, 0)),
        out_shape=jax.ShapeDtypeStruct((N_PAD, d_out), jnp.float32),
    )(p, deg, h, Wl, Wr, b)


def kernel(x, edge_index, Wl1, Wr1, b1, Wl2, Wr2, b2, Wl3, Wr3, b3):
    src = edge_index[0].astype(jnp.int32).reshape(NW, N_CHUNKS, CHUNK)
    dst = edge_index[1].astype(jnp.int32).reshape(NW, N_CHUNKS, CHUNK)
    xp = jnp.pad(x, ((0, N_PAD - N_NODES), (0, 0)))
    z = jnp.zeros((N_PAD, D_FEAT), jnp.float32)
    ones = jnp.ones((CHUNK, D_FEAT), jnp.float32)

    hist = _sc_degree(ones, dst, z)
    p1 = _sc_aggregate(xp, src, dst, z)
    h1, deg = _tc_layer1(p1, hist, xp, Wl1, Wr1, b1)
    p2 = _sc_aggregate(h1, src, dst, z)
    h2 = _tc_layerN(p2, deg, h1, Wl2, Wr2, b2, "relu")
    p3 = _sc_aggregate(h2, src, dst, z)
    return _tc_layerN(p3, deg, h2, Wl3, Wr3, b3, "sigmoid")[:N_NODES]
